# Initial kernel scaffold; baseline (speedup 1.0000x reference)
import jax
import jax.numpy as jnp
from jax.experimental import pallas as pl


def kernel(x, table, W1, b1, W2, b2):
    raise NotImplementedError("write your pallas kernel here")


# speedup vs baseline: 1.5156x; 1.5156x over previous
import functools

import jax
import jax.numpy as jnp
from jax.experimental import pallas as pl
from jax.experimental.pallas import tpu as pltpu
from jax.experimental.pallas import tpu_sc as plsc

VOCAB = 100000
EMBED = 64
WINDOW = 5
HIDDEN = 128
OUT = 1000
BATCH = 16384
CONCAT = WINDOW * EMBED
NUM_IDX = BATCH * WINDOW
WIDE = 2 * EMBED

GATHER_WINDOW = 256
BATCH_TILE = 512
NB = BATCH // BATCH_TILE


def _sc_gather(table_wide, idx_flat):
    mesh = plsc.VectorSubcoreMesh(core_axis_name="core", subcore_axis_name="subcore")

    @functools.partial(
        pl.kernel,
        out_type=jax.ShapeDtypeStruct((NUM_IDX, WIDE), table_wide.dtype),
        mesh=mesh,
    )
    def gather_kernel(table_hbm, idx_hbm, out_hbm):
        def body(idx_vmem, out_vmem):
            pltpu.sync_copy(table_hbm.at[idx_vmem.at[0]], out_vmem)

        pltpu.emit_pipeline(
            body,
            grid=(NUM_IDX // GATHER_WINDOW,),
            in_specs=[
                pl.BlockSpec((1, GATHER_WINDOW), index_map=lambda i: (0, i))
            ],
            out_specs=[
                pl.BlockSpec((GATHER_WINDOW, WIDE), index_map=lambda i: (i, 0))
            ],
            core_axis_name=("core", "subcore"),
            dimension_semantics=(pltpu.PARALLEL,),
        )(idx_hbm, out_hbm)

    return gather_kernel(table_wide, idx_flat.reshape(1, NUM_IDX))


def _mlp_body(w0, w1, w2, w3, w4, par_ref, w1_ref, b1_ref, w2_ref, b2_ref,
              out_ref):
    wides = (w0, w1, w2, w3, w4)
    par = par_ref[...]
    acc = b1_ref[...].astype(jnp.float32)
    for w in range(WINDOW):
        wide = wides[w][...]
        left = wide[:, :EMBED]
        right = wide[:, EMBED:]
        p = jax.lax.broadcast_in_dim(par[:, w], (par.shape[0], EMBED), (0,))
        chosen = left + p * (right - left)
        acc = acc + jax.lax.dot_general(
            chosen, w1_ref[w * EMBED:(w + 1) * EMBED, :],
            (((1,), (0,)), ((), ())),
            precision=jax.lax.Precision.HIGHEST,
            preferred_element_type=jnp.float32,
        )
    h = jnp.tanh(acc)
    o = jax.lax.dot_general(
        h, w2_ref[...],
        (((1,), (0,)), ((), ())),
        precision=jax.lax.Precision.HIGHEST,
        preferred_element_type=jnp.float32,
    ) + b2_ref[...]
    m = jnp.max(o, axis=1, keepdims=True)
    e = jnp.exp(o - m)
    out_ref[...] = e / jnp.sum(e, axis=1, keepdims=True)


def _tc_mlp(wide_t, par8, W1, b1, W2, b2):
    wide_spec = lambda w: pl.BlockSpec(
        (BATCH_TILE, WIDE), functools.partial(lambda w, i: (w * NB + i, 0), w))
    return pl.pallas_call(
        _mlp_body,
        grid=(NB,),
        in_specs=[wide_spec(0), wide_spec(1), wide_spec(2), wide_spec(3),
                  wide_spec(4),
                  pl.BlockSpec((BATCH_TILE, 8), lambda i: (i, 0)),
                  pl.BlockSpec((CONCAT, HIDDEN), lambda i: (0, 0)),
                  pl.BlockSpec((1, HIDDEN), lambda i: (0, 0)),
                  pl.BlockSpec((HIDDEN, OUT), lambda i: (0, 0)),
                  pl.BlockSpec((1, OUT), lambda i: (0, 0))],
        out_specs=pl.BlockSpec((BATCH_TILE, OUT), lambda i: (i, 0)),
        out_shape=jax.ShapeDtypeStruct((BATCH, OUT), jnp.float32),
    )(wide_t, wide_t, wide_t, wide_t, wide_t, par8,
      W1, b1.reshape(1, HIDDEN), W2, b2.reshape(1, OUT))


def kernel(x, table, W1, b1, W2, b2):
    table_wide = table.reshape(VOCAB // 2, WIDE)
    xt = x.T
    idx_wide = (xt // 2).reshape(-1)
    par8 = jnp.pad((x % 2).astype(jnp.float32), ((0, 0), (0, 8 - WINDOW)))
    wide_t = _sc_gather(table_wide, idx_wide)
    return _tc_mlp(wide_t, par8, W1, b1, W2, b2)


# speedup vs baseline: 2.2071x; 1.4562x over previous
import functools

import jax
import jax.numpy as jnp
from jax.experimental import pallas as pl
from jax.experimental.pallas import tpu as pltpu
from jax.experimental.pallas import tpu_sc as plsc

VOCAB = 100000
EMBED = 64
WINDOW = 5
HIDDEN = 128
OUT = 1000
BATCH = 16384
CONCAT = WINDOW * EMBED
NUM_IDX = BATCH * WINDOW
WIDE = 2 * EMBED

GATHER_WINDOW = 256
BATCH_TILE = 1024
NB = BATCH // BATCH_TILE


def _sc_gather(table_wide, idx_flat):
    mesh = plsc.VectorSubcoreMesh(core_axis_name="core", subcore_axis_name="subcore")

    @functools.partial(
        pl.kernel,
        out_type=jax.ShapeDtypeStruct((NUM_IDX, WIDE), table_wide.dtype),
        mesh=mesh,
    )
    def gather_kernel(table_hbm, idx_hbm, out_hbm):
        def body(idx_vmem, out_vmem):
            pltpu.sync_copy(table_hbm.at[idx_vmem.at[0]], out_vmem)

        pltpu.emit_pipeline(
            body,
            grid=(NUM_IDX // GATHER_WINDOW,),
            in_specs=[
                pl.BlockSpec((1, GATHER_WINDOW), index_map=lambda i: (0, i))
            ],
            out_specs=[
                pl.BlockSpec((GATHER_WINDOW, WIDE), index_map=lambda i: (i, 0))
            ],
            core_axis_name=("core", "subcore"),
            dimension_semantics=(pltpu.PARALLEL,),
        )(idx_hbm, out_hbm)

    return gather_kernel(table_wide, idx_flat.reshape(1, NUM_IDX))


def _mlp_body(w0, w1, w2, w3, w4, par_ref, w1_ref, b1_ref, w2_ref, b2_ref,
              out_ref):
    wides = (w0, w1, w2, w3, w4)
    par = par_ref[...]
    acc = b1_ref[...].astype(jnp.float32)
    for w in range(WINDOW):
        wide = wides[w][...]
        left = wide[:, :EMBED]
        right = wide[:, EMBED:]
        p = jax.lax.broadcast_in_dim(par[:, w], (par.shape[0], EMBED), (0,))
        chosen = left + p * (right - left)
        acc = acc + jax.lax.dot_general(
            chosen, w1_ref[w * EMBED:(w + 1) * EMBED, :],
            (((1,), (0,)), ((), ())),
            precision=jax.lax.Precision.DEFAULT,
            preferred_element_type=jnp.float32,
        )
    h = jnp.tanh(acc)
    o = jax.lax.dot_general(
        h, w2_ref[...],
        (((1,), (0,)), ((), ())),
        precision=jax.lax.Precision.DEFAULT,
        preferred_element_type=jnp.float32,
    ) + b2_ref[...]
    m = jnp.max(o, axis=1, keepdims=True)
    e = jnp.exp(o - m)
    out_ref[...] = e / jnp.sum(e, axis=1, keepdims=True)


def _tc_mlp(wide_t, par8, W1, b1, W2, b2):
    wide_spec = lambda w: pl.BlockSpec(
        (BATCH_TILE, WIDE), functools.partial(lambda w, i: (w * NB + i, 0), w))
    return pl.pallas_call(
        _mlp_body,
        grid=(NB,),
        in_specs=[wide_spec(0), wide_spec(1), wide_spec(2), wide_spec(3),
                  wide_spec(4),
                  pl.BlockSpec((BATCH_TILE, 8), lambda i: (i, 0)),
                  pl.BlockSpec((CONCAT, HIDDEN), lambda i: (0, 0)),
                  pl.BlockSpec((1, HIDDEN), lambda i: (0, 0)),
                  pl.BlockSpec((HIDDEN, OUT), lambda i: (0, 0)),
                  pl.BlockSpec((1, OUT), lambda i: (0, 0))],
        out_specs=pl.BlockSpec((BATCH_TILE, OUT), lambda i: (i, 0)),
        out_shape=jax.ShapeDtypeStruct((BATCH, OUT), jnp.float32),
    )(wide_t, wide_t, wide_t, wide_t, wide_t, par8,
      W1, b1.reshape(1, HIDDEN), W2, b2.reshape(1, OUT))


def kernel(x, table, W1, b1, W2, b2):
    table_wide = table.reshape(VOCAB // 2, WIDE)
    xt = x.T
    idx_wide = (xt // 2).reshape(-1)
    par8 = jnp.pad((x % 2).astype(jnp.float32), ((0, 0), (0, 8 - WINDOW)))
    wide_t = _sc_gather(table_wide, idx_wide)
    return _tc_mlp(wide_t, par8, W1, b1, W2, b2)


# speedup vs baseline: 3.4675x; 1.5711x over previous
import functools

import jax
import jax.numpy as jnp
from jax.experimental import pallas as pl
from jax.experimental.pallas import tpu as pltpu
from jax.experimental.pallas import tpu_sc as plsc

VOCAB = 100000
EMBED = 64
WINDOW = 5
HIDDEN = 128
OUT = 1000
BATCH = 16384
CONCAT = WINDOW * EMBED
NUM_IDX = BATCH * WINDOW
WIDE = 2 * EMBED

GATHER_WINDOW = 256
BATCH_TILE = 1024
NB = BATCH // BATCH_TILE


def _sc_gather(table_pad, idx_flat):
    mesh = plsc.VectorSubcoreMesh(core_axis_name="core", subcore_axis_name="subcore")

    @functools.partial(
        pl.kernel,
        out_type=jax.ShapeDtypeStruct((NUM_IDX, WIDE), table_pad.dtype),
        mesh=mesh,
    )
    def gather_kernel(table_hbm, idx_hbm, out_hbm):
        def body(idx_vmem, out_vmem):
            pltpu.sync_copy(table_hbm.at[idx_vmem.at[0]], out_vmem)

        pltpu.emit_pipeline(
            body,
            grid=(NUM_IDX // GATHER_WINDOW,),
            in_specs=[
                pl.BlockSpec((1, GATHER_WINDOW), index_map=lambda i: (0, i))
            ],
            out_specs=[
                pl.BlockSpec((GATHER_WINDOW, WIDE), index_map=lambda i: (i, 0))
            ],
            core_axis_name=("core", "subcore"),
            dimension_semantics=(pltpu.PARALLEL,),
        )(idx_hbm, out_hbm)

    return gather_kernel(table_pad, idx_flat.reshape(1, NUM_IDX))


def _mlp_body(w0, w1, w2, w3, w4, w1_ref, b1_ref, w2_ref, b2t_ref, out_ref):
    wides = (w0, w1, w2, w3, w4)
    acc = b1_ref[...].astype(jnp.float32)
    for w in range(WINDOW):
        acc = acc + jax.lax.dot_general(
            wides[w][:, :EMBED], w1_ref[w * EMBED:(w + 1) * EMBED, :],
            (((1,), (0,)), ((), ())),
            precision=jax.lax.Precision.DEFAULT,
            preferred_element_type=jnp.float32,
        )
    h = jnp.tanh(acc)
    ot = jax.lax.dot_general(
        w2_ref[...], h,
        (((0,), (1,)), ((), ())),
        precision=jax.lax.Precision.DEFAULT,
        preferred_element_type=jnp.float32,
    ) + b2t_ref[...]
    m = jnp.max(ot, axis=0, keepdims=True)
    e = jnp.exp(ot - m)
    out_ref[...] = e / jnp.sum(e, axis=0, keepdims=True)


def _tc_mlp(wide_t, W1, b1, W2, b2):
    wide_spec = lambda w: pl.BlockSpec(
        (BATCH_TILE, WIDE), functools.partial(lambda w, i: (w * NB + i, 0), w))
    return pl.pallas_call(
        _mlp_body,
        grid=(NB,),
        in_specs=[wide_spec(0), wide_spec(1), wide_spec(2), wide_spec(3),
                  wide_spec(4),
                  pl.BlockSpec((CONCAT, HIDDEN), lambda i: (0, 0)),
                  pl.BlockSpec((1, HIDDEN), lambda i: (0, 0)),
                  pl.BlockSpec((HIDDEN, OUT), lambda i: (0, 0)),
                  pl.BlockSpec((OUT, 1), lambda i: (0, 0))],
        out_specs=pl.BlockSpec((OUT, BATCH_TILE), lambda i: (0, i)),
        out_shape=jax.ShapeDtypeStruct((OUT, BATCH), jnp.float32),
    )(wide_t, wide_t, wide_t, wide_t, wide_t,
      W1, b1.reshape(1, HIDDEN), W2, b2.reshape(OUT, 1))


def kernel(x, table, W1, b1, W2, b2):
    table_pad = jnp.pad(table, ((0, 0), (0, WIDE - EMBED)))
    idx = x.T.reshape(-1)
    wide_t = _sc_gather(table_pad, idx)
    return _tc_mlp(wide_t, W1, b1, W2, b2).T


# speedup vs baseline: 3.5301x; 1.0181x over previous
import functools

import jax
import jax.numpy as jnp
from jax.experimental import pallas as pl
from jax.experimental.pallas import tpu as pltpu
from jax.experimental.pallas import tpu_sc as plsc

VOCAB = 100000
EMBED = 64
WINDOW = 5
HIDDEN = 128
OUT = 1000
BATCH = 16384
CONCAT = WINDOW * EMBED
WIDE = 2 * EMBED

NCHUNK = 2
CHUNK = BATCH // NCHUNK
CHUNK_IDX = CHUNK * WINDOW

GATHER_WINDOW = 256
BATCH_TILE = 1024
NBC = CHUNK // BATCH_TILE


def _sc_gather(table_pad, idx_flat):
    mesh = plsc.VectorSubcoreMesh(core_axis_name="core", subcore_axis_name="subcore")

    @functools.partial(
        pl.kernel,
        out_type=jax.ShapeDtypeStruct((CHUNK_IDX, WIDE), table_pad.dtype),
        mesh=mesh,
    )
    def gather_kernel(table_hbm, idx_hbm, out_hbm):
        def body(idx_vmem, out_vmem):
            pltpu.sync_copy(table_hbm.at[idx_vmem.at[0]], out_vmem)

        pltpu.emit_pipeline(
            body,
            grid=(CHUNK_IDX // GATHER_WINDOW,),
            in_specs=[
                pl.BlockSpec((1, GATHER_WINDOW), index_map=lambda i: (0, i))
            ],
            out_specs=[
                pl.BlockSpec((GATHER_WINDOW, WIDE), index_map=lambda i: (i, 0))
            ],
            core_axis_name=("core", "subcore"),
            dimension_semantics=(pltpu.PARALLEL,),
        )(idx_hbm, out_hbm)

    return gather_kernel(table_pad, idx_flat.reshape(1, CHUNK_IDX))


def _mlp_compute(wides, w1_ref, b1_ref, w2_ref, b2t_ref, out_ref):
    acc = b1_ref[...].astype(jnp.float32)
    for w in range(WINDOW):
        acc = acc + jax.lax.dot_general(
            wides[w][:, :EMBED], w1_ref[w * EMBED:(w + 1) * EMBED, :],
            (((1,), (0,)), ((), ())),
            precision=jax.lax.Precision.DEFAULT,
            preferred_element_type=jnp.float32,
        )
    h = jnp.tanh(acc)
    ot = jax.lax.dot_general(
        w2_ref[...], h,
        (((0,), (1,)), ((), ())),
        precision=jax.lax.Precision.DEFAULT,
        preferred_element_type=jnp.float32,
    ) + b2t_ref[...]
    m = jnp.max(ot, axis=0, keepdims=True)
    e = jnp.exp(ot - m)
    out_ref[...] = e / jnp.sum(e, axis=0, keepdims=True)


def _mlp_body(w0, w1, w2, w3, w4, w1_ref, b1_ref, w2_ref, b2t_ref, out_ref):
    _mlp_compute((w0, w1, w2, w3, w4), w1_ref, b1_ref, w2_ref, b2t_ref, out_ref)


def _mlp_body_alias(prev_ref, w0, w1, w2, w3, w4, w1_ref, b1_ref, w2_ref,
                    b2t_ref, out_ref):
    del prev_ref
    _mlp_compute((w0, w1, w2, w3, w4), w1_ref, b1_ref, w2_ref, b2t_ref, out_ref)


def _tc_mlp_chunk(c, prev, wide_c, W1, b1, W2, b2):
    wide_spec = lambda w: pl.BlockSpec(
        (BATCH_TILE, WIDE), functools.partial(lambda w, i: (w * NBC + i, 0), w))
    weight_specs = [
        pl.BlockSpec((CONCAT, HIDDEN), lambda i: (0, 0)),
        pl.BlockSpec((1, HIDDEN), lambda i: (0, 0)),
        pl.BlockSpec((HIDDEN, OUT), lambda i: (0, 0)),
        pl.BlockSpec((OUT, 1), lambda i: (0, 0)),
    ]
    out_spec = pl.BlockSpec(
        (OUT, BATCH_TILE), functools.partial(lambda c, i: (0, c * NBC + i), c))
    wide_args = (wide_c,) * WINDOW
    weight_args = (W1, b1.reshape(1, HIDDEN), W2, b2.reshape(OUT, 1))
    if c == 0:
        return pl.pallas_call(
            _mlp_body,
            grid=(NBC,),
            in_specs=[wide_spec(w) for w in range(WINDOW)] + weight_specs,
            out_specs=out_spec,
            out_shape=jax.ShapeDtypeStruct((OUT, BATCH), jnp.float32),
        )(*wide_args, *weight_args)
    return pl.pallas_call(
        _mlp_body_alias,
        grid=(NBC,),
        in_specs=[pl.BlockSpec(memory_space=pl.ANY)]
        + [wide_spec(w) for w in range(WINDOW)] + weight_specs,
        out_specs=out_spec,
        out_shape=jax.ShapeDtypeStruct((OUT, BATCH), jnp.float32),
        input_output_aliases={0: 0},
    )(prev, *wide_args, *weight_args)


def kernel(x, table, W1, b1, W2, b2):
    table_pad = jnp.pad(table, ((0, 0), (0, WIDE - EMBED)))
    xt = x.T
    out = None
    for c in range(NCHUNK):
        idx_c = xt[:, c * CHUNK:(c + 1) * CHUNK].reshape(-1)
        wide_c = _sc_gather(table_pad, idx_c)
        out = _tc_mlp_chunk(c, out, wide_c, W1, b1, W2, b2)
    return out.T
